# baseline (speedup 1.0000x reference)
import functools

import jax
import jax.numpy as jnp
from jax import lax
from jax.experimental import pallas as pl
from jax.experimental.pallas import tpu as pltpu
from jax.experimental.pallas import tpu_sc as plsc

N = 10000
E = 320000
D_IN = 128
DH = 128
DQ = 64
DHID = 256
NG = 64
NC = 2
NP = 2
NS = 16
CH = 128
RING = 4
AGG_ROWS = 10240
TILE_ROWS = AGG_ROWS // NS
WB = TILE_ROWS // CH
RB = 1000

NCH1 = 80
NCHM = 160


def _make_agg(table_rows, n_chunks):
    mesh = plsc.VectorSubcoreMesh(core_axis_name="c", subcore_axis_name="s")

    @functools.partial(
        pl.kernel,
        mesh=mesh,
        compiler_params=pltpu.CompilerParams(use_tc_tiling_on_sc=False),
        out_type=jax.ShapeDtypeStruct((NC, NP, AGG_ROWS, DQ), jnp.float32),
        scratch_types=[
            pltpu.VMEM((n_chunks + RING, CH), jnp.int32),
            pltpu.VMEM((n_chunks, CH), jnp.int32),
            pltpu.VMEM((RING, CH, DQ), jnp.float32),
        ]
        + [pltpu.SemaphoreType.DMA] * (2 * RING)
        + [pltpu.VMEM_SHARED((AGG_ROWS, DQ), jnp.float32)],
    )
    def agg(table, srcs, dsts, out, src_v, dst_v, rows, *rest):
        semg = rest[:RING]
        sems = rest[RING:2 * RING]
        agg_sh = rest[2 * RING]
        c = lax.axis_index("c")
        s = lax.axis_index("s")

        pltpu.sync_copy(dsts.at[c, s], dst_v)

        for p in range(NP):
            def zrow(i, carry):
                for k in range(DQ // 16):
                    rows[0, i, pl.ds(k * 16, 16)] = jnp.zeros(
                        (16,), jnp.float32)
                return carry

            lax.fori_loop(0, CH, zrow, 0)
            for k in range(WB):
                pltpu.sync_copy(
                    rows.at[0],
                    agg_sh.at[pl.ds(s * TILE_ROWS + k * CH, CH)])
            plsc.subcore_barrier()

            pltpu.sync_copy(srcs.at[c, p, s], src_v)

            for b in range(RING):
                pltpu.async_copy(table.at[src_v.at[b]], rows.at[b], semg[b])

            def body(i, carry):
                j0 = RING * i
                for b in range(RING):
                    pltpu.make_async_copy(
                        table.at[src_v.at[j0 + b]], rows.at[b],
                        semg[b]).wait()
                    pltpu.async_copy(
                        rows.at[b], agg_sh.at[dst_v.at[j0 + b]], sems[b],
                        add=True)
                for b in range(RING):
                    pltpu.make_async_copy(
                        rows.at[b], agg_sh.at[dst_v.at[j0 + b]],
                        sems[b]).wait()
                    pltpu.async_copy(
                        table.at[src_v.at[j0 + RING + b]], rows.at[b],
                        semg[b])
                return carry

            lax.fori_loop(0, n_chunks // RING, body, 0)
            for b in range(RING):
                pltpu.make_async_copy(
                    table.at[src_v.at[n_chunks + b]], rows.at[b],
                    semg[b]).wait()
            plsc.subcore_barrier()

            for k in range(WB):
                off = s * TILE_ROWS + k * CH
                pltpu.sync_copy(agg_sh.at[pl.ds(off, CH)], rows.at[0])
                pltpu.sync_copy(rows.at[0], out.at[c, p].at[pl.ds(off, CH)])

    return agg


_agg_l1 = _make_agg(NP * N, NCH1)
_agg_mid = _make_agg(NC * NP * N, NCHM)


def _mlp_l1_body(x_ref, agg_ref, sc_ref, w1_ref, b1_ref, w2_ref, b2_ref, out_ref):
    a0 = jnp.concatenate([agg_ref[0, 0], agg_ref[0, 1]], axis=1)
    a1 = jnp.concatenate([agg_ref[1, 0], agg_ref[1, 1]], axis=1)
    z = x_ref[...] * sc_ref[0, 0] + a0 + a1
    y = jnp.dot(z, w1_ref[...], preferred_element_type=jnp.float32) + b1_ref[...]
    y = jnp.maximum(y, 0.0)
    o = jnp.dot(y, w2_ref[...], preferred_element_type=jnp.float32) + b2_ref[...]
    out_ref[0] = o[:, :DH]
    out_ref[1] = o[:, DH:]


def _mlp_mid_body(h_ref, agg_ref, sc_ref, w1_ref, b1_ref, w2_ref, b2_ref, out_ref):
    hcat = jnp.concatenate([h_ref[0], h_ref[1]], axis=1)
    acat = jnp.concatenate(
        [agg_ref[0, 0], agg_ref[0, 1], agg_ref[1, 0], agg_ref[1, 1]], axis=1)
    z = hcat * sc_ref[0, 0] + acat
    y = jnp.dot(z, w1_ref[...], preferred_element_type=jnp.float32) + b1_ref[...]
    y = jnp.maximum(y, 0.0)
    o = jnp.dot(y, w2_ref[...], preferred_element_type=jnp.float32) + b2_ref[...]
    out_ref[0] = o[:, :DH]
    out_ref[1] = o[:, DH:]


def _mlp_l1(x, agg, sc, w1, b1, w2, b2):
    return pl.pallas_call(
        _mlp_l1_body,
        grid=(N // RB,),
        in_specs=[
            pl.BlockSpec((RB, D_IN), lambda i: (i, 0)),
            pl.BlockSpec((NC, NP, RB, DQ), lambda i: (0, 0, i, 0)),
            pl.BlockSpec((1, 1), lambda i: (0, 0)),
            pl.BlockSpec((D_IN, DHID), lambda i: (0, 0)),
            pl.BlockSpec((1, DHID), lambda i: (0, 0)),
            pl.BlockSpec((DHID, DHID), lambda i: (0, 0)),
            pl.BlockSpec((1, DHID), lambda i: (0, 0)),
        ],
        out_specs=pl.BlockSpec((NC, RB, DH), lambda i: (0, i, 0)),
        out_shape=jax.ShapeDtypeStruct((NC, N, DH), jnp.float32),
    )(x, agg, sc, w1, b1, w2, b2)


def _mlp_mid(h, agg, sc, w1, b1, w2, b2):
    return pl.pallas_call(
        _mlp_mid_body,
        grid=(N // RB,),
        in_specs=[
            pl.BlockSpec((NC, RB, DH), lambda i: (0, i, 0)),
            pl.BlockSpec((NC, NP, RB, DQ), lambda i: (0, 0, i, 0)),
            pl.BlockSpec((1, 1), lambda i: (0, 0)),
            pl.BlockSpec((DHID, DHID), lambda i: (0, 0)),
            pl.BlockSpec((1, DHID), lambda i: (0, 0)),
            pl.BlockSpec((DHID, DHID), lambda i: (0, 0)),
            pl.BlockSpec((1, DHID), lambda i: (0, 0)),
        ],
        out_specs=pl.BlockSpec((NC, RB, DH), lambda i: (0, i, 0)),
        out_shape=jax.ShapeDtypeStruct((NC, N, DH), jnp.float32),
    )(h, agg, sc, w1, b1, w2, b2)


def _pool_body(h_ref, b_ref, wo_ref, bo_ref, out_ref, acc_ref):
    i = pl.program_id(0)

    @pl.when(i == 0)
    def _():
        acc_ref[...] = jnp.zeros_like(acc_ref)

    hcat = jnp.concatenate([h_ref[0], h_ref[1]], axis=1)
    oh = (b_ref[...] == lax.broadcasted_iota(jnp.int32, (RB, NG), 1))
    oh = oh.astype(jnp.float32)
    acc_ref[...] += lax.dot_general(
        oh, hcat, (((0,), (0,)), ((), ())), preferred_element_type=jnp.float32)

    @pl.when(i == pl.num_programs(0) - 1)
    def _():
        out_ref[...] = jnp.dot(
            acc_ref[...], wo_ref[...], preferred_element_type=jnp.float32
        ) + bo_ref[...]


def _pool(h, batch2d, wout, bout):
    return pl.pallas_call(
        _pool_body,
        grid=(N // RB,),
        in_specs=[
            pl.BlockSpec((NC, RB, DH), lambda i: (0, i, 0)),
            pl.BlockSpec((RB, 1), lambda i: (i, 0)),
            pl.BlockSpec((DHID, 1), lambda i: (0, 0)),
            pl.BlockSpec((1, 1), lambda i: (0, 0)),
        ],
        out_specs=pl.BlockSpec((NG, 1), lambda i: (0, 0)),
        out_shape=jax.ShapeDtypeStruct((NG, 1), jnp.float32),
        scratch_shapes=[pltpu.VMEM((NG, DHID), jnp.float32)],
    )(h, batch2d, wout, bout)


def kernel(x, edge_index, batch, params):
    src = edge_index[0].astype(jnp.int32)
    dst = edge_index[1].astype(jnp.int32)

    e1 = E // NC
    t1 = NS * NCH1 * CH
    p1 = t1 - e1
    pad_src1 = jnp.arange(p1, dtype=jnp.int32) % N
    pad_dst1 = N + jnp.arange(p1, dtype=jnp.int32) % (AGG_ROWS - N)
    src1h = [jnp.concatenate([src[:e1], pad_src1]),
             jnp.concatenate([src[e1:], pad_src1])]
    src1 = jnp.stack(
        [jnp.stack([2 * src1h[c] + p for p in range(NP)]) for c in range(NC)]
    ).reshape(NC, NP, NS, NCH1, CH)
    src1 = jnp.concatenate(
        [src1, jnp.zeros((NC, NP, NS, RING, CH), jnp.int32)], axis=3)
    dst1 = jnp.stack([
        jnp.concatenate([dst[:e1], pad_dst1]),
        jnp.concatenate([dst[e1:], pad_dst1]),
    ]).reshape(NC, NS, NCH1, CH)

    tm = NS * NCHM * CH
    pm = tm - E
    pad_srcm = jnp.arange(pm, dtype=jnp.int32) % N
    pad_dstm = N + jnp.arange(pm, dtype=jnp.int32) % (AGG_ROWS - N)
    srcm_base = jnp.concatenate([src, pad_srcm])
    srcm = jnp.stack(
        [jnp.stack([2 * (srcm_base + c * N) + p for p in range(NP)])
         for c in range(NC)]
    ).reshape(NC, NP, NS, NCHM, CH)
    srcm = jnp.concatenate(
        [srcm, jnp.zeros((NC, NP, NS, RING, CH), jnp.int32)], axis=3)
    dstm_1 = jnp.concatenate([dst, pad_dstm])
    dstm = jnp.stack([dstm_1, dstm_1]).reshape(NC, NS, NCHM, CH)

    batch2d = batch.astype(jnp.int32).reshape(N, 1)

    p = params['layers'][0]
    sc = (1.0 + p['eps']).reshape(1, 1).astype(jnp.float32)
    agg = _agg_l1(x.reshape(NP * N, DQ), src1, dst1)
    h = _mlp_l1(x, agg, sc, p['W1'], p['b1'].reshape(1, DHID),
                p['W2'], p['b2'].reshape(1, DHID))

    for p in params['layers'][1:]:
        sc = (1.0 + p['eps']).reshape(1, 1).astype(jnp.float32)
        table = h.reshape(NC * NP * N, DQ)
        agg = _agg_mid(table, srcm, dstm)
        h = _mlp_mid(h, agg, sc, p['W1'], p['b1'].reshape(1, DHID),
                     p['W2'], p['b2'].reshape(1, DHID))

    return _pool(h, batch2d, params['Wout'], params['bout'].reshape(1, 1))


# speedup vs baseline: 1.9637x; 1.9637x over previous
import functools

import jax
import jax.numpy as jnp
from jax import lax
from jax.experimental import pallas as pl
from jax.experimental.pallas import tpu as pltpu
from jax.experimental.pallas import tpu_sc as plsc

N = 10000
E = 320000
D_IN = 128
DH = 128
DQ = 64
DHID = 256
NG = 64
NC = 2
NP = 2
NS = 16
CH = 128
AGG_ROWS = 10240
TILE_ROWS = AGG_ROWS // NS
WB = TILE_ROWS // CH
RB = 1000

NCH1 = 80
NCHM = 158


def _make_agg(table_rows, n_chunks):
    mesh = plsc.VectorSubcoreMesh(core_axis_name="c", subcore_axis_name="s")

    @functools.partial(
        pl.kernel,
        mesh=mesh,
        compiler_params=pltpu.CompilerParams(use_tc_tiling_on_sc=False),
        out_type=jax.ShapeDtypeStruct((NC, NP, AGG_ROWS, DQ), jnp.float32),
        scratch_types=[
            pltpu.VMEM((n_chunks + 1, CH), jnp.int32),
            pltpu.VMEM((n_chunks, CH), jnp.int32),
            pltpu.VMEM((2, CH, DQ), jnp.float32),
            pltpu.VMEM_SHARED((AGG_ROWS, DQ), jnp.float32),
            pltpu.SemaphoreType.DMA,
            pltpu.SemaphoreType.DMA,
        ],
    )
    def agg(table, srcs, dsts, out, src_v, dst_v, rows, agg_sh, sem0, sem1):
        c = lax.axis_index("c")
        s = lax.axis_index("s")

        pltpu.sync_copy(dsts.at[c, s], dst_v)

        for p in range(NP):
            def zrow(i, carry):
                for k in range(DQ // 16):
                    rows[0, i, pl.ds(k * 16, 16)] = jnp.zeros(
                        (16,), jnp.float32)
                return carry

            lax.fori_loop(0, CH, zrow, 0)
            for k in range(WB):
                pltpu.sync_copy(
                    rows.at[0], agg_sh.at[pl.ds(s * TILE_ROWS + k * CH, CH)])
            plsc.subcore_barrier()

            pltpu.sync_copy(srcs.at[c, p, s], src_v)

            pltpu.async_copy(table.at[src_v.at[0]], rows.at[0], sem0)

            def body(i, carry):
                j0 = 2 * i
                pltpu.async_copy(table.at[src_v.at[j0 + 1]], rows.at[1], sem1)
                pltpu.make_async_copy(
                    table.at[src_v.at[j0]], rows.at[0], sem0).wait()
                pltpu.sync_copy(rows.at[0], agg_sh.at[dst_v.at[j0]], add=True)
                pltpu.async_copy(table.at[src_v.at[j0 + 2]], rows.at[0], sem0)
                pltpu.make_async_copy(
                    table.at[src_v.at[j0 + 1]], rows.at[1], sem1).wait()
                pltpu.sync_copy(
                    rows.at[1], agg_sh.at[dst_v.at[j0 + 1]], add=True)
                return carry

            lax.fori_loop(0, n_chunks // 2, body, 0)
            pltpu.make_async_copy(
                table.at[src_v.at[n_chunks]], rows.at[0], sem0).wait()
            plsc.subcore_barrier()

            for k in range(WB):
                off = s * TILE_ROWS + k * CH
                pltpu.sync_copy(agg_sh.at[pl.ds(off, CH)], rows.at[0])
                pltpu.sync_copy(rows.at[0], out.at[c, p].at[pl.ds(off, CH)])

    return agg


_agg_l1 = _make_agg(NP * N, NCH1)
_agg_mid = _make_agg(NC * NP * N, NCHM)


def _mlp_l1_body(x_ref, agg_ref, sc_ref, w1_ref, b1_ref, w2_ref, b2_ref, out_ref):
    a0 = jnp.concatenate([agg_ref[0, 0], agg_ref[0, 1]], axis=1)
    a1 = jnp.concatenate([agg_ref[1, 0], agg_ref[1, 1]], axis=1)
    z = x_ref[...] * sc_ref[0, 0] + a0 + a1
    y = jnp.dot(z, w1_ref[...], preferred_element_type=jnp.float32) + b1_ref[...]
    y = jnp.maximum(y, 0.0)
    o = jnp.dot(y, w2_ref[...], preferred_element_type=jnp.float32) + b2_ref[...]
    for c in range(NC):
        for p in range(NP):
            out_ref[c, p] = o[:, (c * NP + p) * DQ:(c * NP + p + 1) * DQ]


def _mlp_mid_body(h_ref, agg_ref, sc_ref, w1_ref, b1_ref, w2_ref, b2_ref, out_ref):
    hcat = jnp.concatenate(
        [h_ref[0, 0], h_ref[0, 1], h_ref[1, 0], h_ref[1, 1]], axis=1)
    acat = jnp.concatenate(
        [agg_ref[0, 0], agg_ref[0, 1], agg_ref[1, 0], agg_ref[1, 1]], axis=1)
    z = hcat * sc_ref[0, 0] + acat
    y = jnp.dot(z, w1_ref[...], preferred_element_type=jnp.float32) + b1_ref[...]
    y = jnp.maximum(y, 0.0)
    o = jnp.dot(y, w2_ref[...], preferred_element_type=jnp.float32) + b2_ref[...]
    for c in range(NC):
        for p in range(NP):
            out_ref[c, p] = o[:, (c * NP + p) * DQ:(c * NP + p + 1) * DQ]


def _mlp_l1(x, agg, sc, w1, b1, w2, b2):
    return pl.pallas_call(
        _mlp_l1_body,
        grid=(N // RB,),
        in_specs=[
            pl.BlockSpec((RB, D_IN), lambda i: (i, 0)),
            pl.BlockSpec((NC, NP, RB, DQ), lambda i: (0, 0, i, 0)),
            pl.BlockSpec((1, 1), lambda i: (0, 0)),
            pl.BlockSpec((D_IN, DHID), lambda i: (0, 0)),
            pl.BlockSpec((1, DHID), lambda i: (0, 0)),
            pl.BlockSpec((DHID, DHID), lambda i: (0, 0)),
            pl.BlockSpec((1, DHID), lambda i: (0, 0)),
        ],
        out_specs=pl.BlockSpec((NC, NP, RB, DQ), lambda i: (0, 0, i, 0)),
        out_shape=jax.ShapeDtypeStruct((NC, NP, N, DQ), jnp.float32),
    )(x, agg, sc, w1, b1, w2, b2)


def _mlp_mid(h, agg, sc, w1, b1, w2, b2):
    return pl.pallas_call(
        _mlp_mid_body,
        grid=(N // RB,),
        in_specs=[
            pl.BlockSpec((NC, NP, RB, DQ), lambda i: (0, 0, i, 0)),
            pl.BlockSpec((NC, NP, RB, DQ), lambda i: (0, 0, i, 0)),
            pl.BlockSpec((1, 1), lambda i: (0, 0)),
            pl.BlockSpec((DHID, DHID), lambda i: (0, 0)),
            pl.BlockSpec((1, DHID), lambda i: (0, 0)),
            pl.BlockSpec((DHID, DHID), lambda i: (0, 0)),
            pl.BlockSpec((1, DHID), lambda i: (0, 0)),
        ],
        out_specs=pl.BlockSpec((NC, NP, RB, DQ), lambda i: (0, 0, i, 0)),
        out_shape=jax.ShapeDtypeStruct((NC, NP, N, DQ), jnp.float32),
    )(h, agg, sc, w1, b1, w2, b2)


def _pool_body(h_ref, b_ref, wo_ref, bo_ref, out_ref, acc_ref):
    i = pl.program_id(0)

    @pl.when(i == 0)
    def _():
        acc_ref[...] = jnp.zeros_like(acc_ref)

    hcat = jnp.concatenate(
        [h_ref[0, 0], h_ref[0, 1], h_ref[1, 0], h_ref[1, 1]], axis=1)
    oh = (b_ref[...] == lax.broadcasted_iota(jnp.int32, (RB, NG), 1))
    oh = oh.astype(jnp.float32)
    acc_ref[...] += lax.dot_general(
        oh, hcat, (((0,), (0,)), ((), ())), preferred_element_type=jnp.float32)

    @pl.when(i == pl.num_programs(0) - 1)
    def _():
        out_ref[...] = jnp.dot(
            acc_ref[...], wo_ref[...], preferred_element_type=jnp.float32
        ) + bo_ref[...]


def _pool(h, batch2d, wout, bout):
    return pl.pallas_call(
        _pool_body,
        grid=(N // RB,),
        in_specs=[
            pl.BlockSpec((NC, NP, RB, DQ), lambda i: (0, 0, i, 0)),
            pl.BlockSpec((RB, 1), lambda i: (i, 0)),
            pl.BlockSpec((DHID, 1), lambda i: (0, 0)),
            pl.BlockSpec((1, 1), lambda i: (0, 0)),
        ],
        out_specs=pl.BlockSpec((NG, 1), lambda i: (0, 0)),
        out_shape=jax.ShapeDtypeStruct((NG, 1), jnp.float32),
        scratch_shapes=[pltpu.VMEM((NG, DHID), jnp.float32)],
    )(h, batch2d, wout, bout)


def kernel(x, edge_index, batch, params):
    src = edge_index[0].astype(jnp.int32)
    dst = edge_index[1].astype(jnp.int32)

    e1 = E // NC
    t1 = NS * NCH1 * CH
    p1 = t1 - e1
    pad_src1 = jnp.arange(p1, dtype=jnp.int32) % N
    pad_dst1 = N + jnp.arange(p1, dtype=jnp.int32) % (AGG_ROWS - N)
    src1h = [jnp.concatenate([src[:e1], pad_src1]),
             jnp.concatenate([src[e1:], pad_src1])]
    src1 = jnp.stack(
        [jnp.stack([src1h[c] + p * N for p in range(NP)]) for c in range(NC)]
    ).reshape(NC, NP, NS, NCH1, CH)
    src1 = jnp.concatenate(
        [src1, jnp.zeros((NC, NP, NS, 1, CH), jnp.int32)], axis=3)
    dst1 = jnp.stack([
        jnp.concatenate([dst[:e1], pad_dst1]),
        jnp.concatenate([dst[e1:], pad_dst1]),
    ]).reshape(NC, NS, NCH1, CH)

    tm = NS * NCHM * CH
    pm = tm - E
    pad_srcm = jnp.arange(pm, dtype=jnp.int32) % N
    pad_dstm = N + jnp.arange(pm, dtype=jnp.int32) % (AGG_ROWS - N)
    srcm_base = jnp.concatenate([src, pad_srcm])
    srcm = jnp.stack(
        [jnp.stack([srcm_base + (c * NP + p) * N for p in range(NP)])
         for c in range(NC)]
    ).reshape(NC, NP, NS, NCHM, CH)
    srcm = jnp.concatenate(
        [srcm, jnp.zeros((NC, NP, NS, 1, CH), jnp.int32)], axis=3)
    dstm_1 = jnp.concatenate([dst, pad_dstm])
    dstm = jnp.stack([dstm_1, dstm_1]).reshape(NC, NS, NCHM, CH)

    batch2d = batch.astype(jnp.int32).reshape(N, 1)

    p = params['layers'][0]
    sc = (1.0 + p['eps']).reshape(1, 1).astype(jnp.float32)
    table1 = jnp.concatenate([x[:, :DQ], x[:, DQ:]], axis=0)
    agg = _agg_l1(table1, src1, dst1)
    h = _mlp_l1(x, agg, sc, p['W1'], p['b1'].reshape(1, DHID),
                p['W2'], p['b2'].reshape(1, DHID))

    for p in params['layers'][1:]:
        sc = (1.0 + p['eps']).reshape(1, 1).astype(jnp.float32)
        table = h.reshape(NC * NP * N, DQ)
        agg = _agg_mid(table, srcm, dstm)
        h = _mlp_mid(h, agg, sc, p['W1'], p['b1'].reshape(1, DHID),
                     p['W2'], p['b2'].reshape(1, DHID))

    return _pool(h, batch2d, params['Wout'], params['bout'].reshape(1, 1))


# speedup vs baseline: 2.1541x; 1.0969x over previous
import functools

import jax
import jax.numpy as jnp
from jax import lax
from jax.experimental import pallas as pl
from jax.experimental.pallas import tpu as pltpu
from jax.experimental.pallas import tpu_sc as plsc

N = 10000
E = 320000
D_IN = 128
DH = 128
DQ = 64
DHID = 256
NG = 64
NC = 2
NP = 2
NS = 16
CH = 128
AGG_ROWS = 10240
TILE_ROWS = AGG_ROWS // NS
WB = TILE_ROWS // CH
RB = 1000

NCH1 = 79
NCHM = 158


def _make_agg(table_rows, n_chunks, pipelined):
    mesh = plsc.VectorSubcoreMesh(core_axis_name="c", subcore_axis_name="s")

    @functools.partial(
        pl.kernel,
        mesh=mesh,
        compiler_params=pltpu.CompilerParams(use_tc_tiling_on_sc=False),
        out_type=jax.ShapeDtypeStruct((NC, NP, AGG_ROWS, DQ), jnp.float32),
        scratch_types=[
            pltpu.VMEM((n_chunks + 1, CH), jnp.int32),
            pltpu.VMEM((n_chunks, CH), jnp.int32),
            pltpu.VMEM((2, CH, DQ), jnp.float32),
            pltpu.VMEM_SHARED((AGG_ROWS, DQ), jnp.float32),
            pltpu.SemaphoreType.DMA,
            pltpu.SemaphoreType.DMA,
        ],
    )
    def agg(table, srcs, dsts, out, src_v, dst_v, rows, agg_sh, sem0, sem1):
        c = lax.axis_index("c")
        s = lax.axis_index("s")

        pltpu.sync_copy(dsts.at[c, s], dst_v)

        for p in range(NP):
            def zrow(i, carry):
                for k in range(DQ // 16):
                    rows[0, i, pl.ds(k * 16, 16)] = jnp.zeros(
                        (16,), jnp.float32)
                return carry

            lax.fori_loop(0, CH, zrow, 0)
            for k in range(WB):
                pltpu.sync_copy(
                    rows.at[0], agg_sh.at[pl.ds(s * TILE_ROWS + k * CH, CH)])
            plsc.subcore_barrier()

            pltpu.sync_copy(srcs.at[c, p, s], src_v)

            if pipelined:
                pltpu.async_copy(table.at[src_v.at[0]], rows.at[0], sem0)

                def body(i, carry):
                    j0 = 2 * i
                    pltpu.async_copy(
                        table.at[src_v.at[j0 + 1]], rows.at[1], sem1)
                    pltpu.make_async_copy(
                        table.at[src_v.at[j0]], rows.at[0], sem0).wait()
                    pltpu.sync_copy(
                        rows.at[0], agg_sh.at[dst_v.at[j0]], add=True)
                    pltpu.async_copy(
                        table.at[src_v.at[j0 + 2]], rows.at[0], sem0)
                    pltpu.make_async_copy(
                        table.at[src_v.at[j0 + 1]], rows.at[1], sem1).wait()
                    pltpu.sync_copy(
                        rows.at[1], agg_sh.at[dst_v.at[j0 + 1]], add=True)
                    return carry

                lax.fori_loop(0, n_chunks // 2, body, 0)
                pltpu.make_async_copy(
                    table.at[src_v.at[n_chunks]], rows.at[0], sem0).wait()
            else:
                def body(j, carry):
                    pltpu.async_copy(
                        table.at[src_v.at[j]], rows.at[0], sem0).wait()
                    pltpu.sync_copy(
                        rows.at[0], agg_sh.at[dst_v.at[j]], add=True)
                    return carry

                lax.fori_loop(0, n_chunks, body, 0)
            plsc.subcore_barrier()

            for k in range(WB):
                off = s * TILE_ROWS + k * CH
                pltpu.sync_copy(agg_sh.at[pl.ds(off, CH)], rows.at[0])
                pltpu.sync_copy(rows.at[0], out.at[c, p].at[pl.ds(off, CH)])

    return agg


_agg_l1 = _make_agg(NP * N, NCH1, pipelined=False)
_agg_mid = _make_agg(NC * NP * N, NCHM, pipelined=True)


def _mlp_l1_body(x_ref, agg_ref, sc_ref, w1_ref, b1_ref, w2_ref, b2_ref, out_ref):
    a0 = jnp.concatenate([agg_ref[0, 0], agg_ref[0, 1]], axis=1)
    a1 = jnp.concatenate([agg_ref[1, 0], agg_ref[1, 1]], axis=1)
    z = x_ref[...] * sc_ref[0, 0] + a0 + a1
    y = jnp.dot(z, w1_ref[...], preferred_element_type=jnp.float32) + b1_ref[...]
    y = jnp.maximum(y, 0.0)
    o = jnp.dot(y, w2_ref[...], preferred_element_type=jnp.float32) + b2_ref[...]
    out_ref[0] = o[:, :DH]
    out_ref[1] = o[:, DH:]


def _mlp_mid_body(h_ref, agg_ref, sc_ref, w1_ref, b1_ref, w2_ref, b2_ref, out_ref):
    hcat = jnp.concatenate([h_ref[0], h_ref[1]], axis=1)
    acat = jnp.concatenate(
        [agg_ref[0, 0], agg_ref[0, 1], agg_ref[1, 0], agg_ref[1, 1]], axis=1)
    z = hcat * sc_ref[0, 0] + acat
    y = jnp.dot(z, w1_ref[...], preferred_element_type=jnp.float32) + b1_ref[...]
    y = jnp.maximum(y, 0.0)
    o = jnp.dot(y, w2_ref[...], preferred_element_type=jnp.float32) + b2_ref[...]
    out_ref[0] = o[:, :DH]
    out_ref[1] = o[:, DH:]


def _mlp_l1(x, agg, sc, w1, b1, w2, b2):
    return pl.pallas_call(
        _mlp_l1_body,
        grid=(N // RB,),
        in_specs=[
            pl.BlockSpec((RB, D_IN), lambda i: (i, 0)),
            pl.BlockSpec((NC, NP, RB, DQ), lambda i: (0, 0, i, 0)),
            pl.BlockSpec((1, 1), lambda i: (0, 0)),
            pl.BlockSpec((D_IN, DHID), lambda i: (0, 0)),
            pl.BlockSpec((1, DHID), lambda i: (0, 0)),
            pl.BlockSpec((DHID, DHID), lambda i: (0, 0)),
            pl.BlockSpec((1, DHID), lambda i: (0, 0)),
        ],
        out_specs=pl.BlockSpec((NC, RB, DH), lambda i: (0, i, 0)),
        out_shape=jax.ShapeDtypeStruct((NC, N, DH), jnp.float32),
    )(x, agg, sc, w1, b1, w2, b2)


def _mlp_mid(h, agg, sc, w1, b1, w2, b2):
    return pl.pallas_call(
        _mlp_mid_body,
        grid=(N // RB,),
        in_specs=[
            pl.BlockSpec((NC, RB, DH), lambda i: (0, i, 0)),
            pl.BlockSpec((NC, NP, RB, DQ), lambda i: (0, 0, i, 0)),
            pl.BlockSpec((1, 1), lambda i: (0, 0)),
            pl.BlockSpec((DHID, DHID), lambda i: (0, 0)),
            pl.BlockSpec((1, DHID), lambda i: (0, 0)),
            pl.BlockSpec((DHID, DHID), lambda i: (0, 0)),
            pl.BlockSpec((1, DHID), lambda i: (0, 0)),
        ],
        out_specs=pl.BlockSpec((NC, RB, DH), lambda i: (0, i, 0)),
        out_shape=jax.ShapeDtypeStruct((NC, N, DH), jnp.float32),
    )(h, agg, sc, w1, b1, w2, b2)


def _pool_body(h_ref, b_ref, wo_ref, bo_ref, out_ref, acc_ref):
    i = pl.program_id(0)

    @pl.when(i == 0)
    def _():
        acc_ref[...] = jnp.zeros_like(acc_ref)

    hcat = jnp.concatenate([h_ref[0], h_ref[1]], axis=1)
    oh = (b_ref[...] == lax.broadcasted_iota(jnp.int32, (RB, NG), 1))
    oh = oh.astype(jnp.float32)
    acc_ref[...] += lax.dot_general(
        oh, hcat, (((0,), (0,)), ((), ())), preferred_element_type=jnp.float32)

    @pl.when(i == pl.num_programs(0) - 1)
    def _():
        out_ref[...] = jnp.dot(
            acc_ref[...], wo_ref[...], preferred_element_type=jnp.float32
        ) + bo_ref[...]


def _pool(h, batch2d, wout, bout):
    return pl.pallas_call(
        _pool_body,
        grid=(N // RB,),
        in_specs=[
            pl.BlockSpec((NC, RB, DH), lambda i: (0, i, 0)),
            pl.BlockSpec((RB, 1), lambda i: (i, 0)),
            pl.BlockSpec((DHID, 1), lambda i: (0, 0)),
            pl.BlockSpec((1, 1), lambda i: (0, 0)),
        ],
        out_specs=pl.BlockSpec((NG, 1), lambda i: (0, 0)),
        out_shape=jax.ShapeDtypeStruct((NG, 1), jnp.float32),
        scratch_shapes=[pltpu.VMEM((NG, DHID), jnp.float32)],
    )(h, batch2d, wout, bout)


def kernel(x, edge_index, batch, params):
    src = edge_index[0].astype(jnp.int32)
    dst = edge_index[1].astype(jnp.int32)

    e1 = E // NC
    t1 = NS * NCH1 * CH
    p1 = t1 - e1
    pad_src1 = jnp.arange(p1, dtype=jnp.int32) % N
    pad_dst1 = N + jnp.arange(p1, dtype=jnp.int32) % (AGG_ROWS - N)
    src1h = [jnp.concatenate([src[:e1], pad_src1]),
             jnp.concatenate([src[e1:], pad_src1])]
    src1 = jnp.stack(
        [jnp.stack([2 * src1h[c] + p for p in range(NP)]) for c in range(NC)]
    ).reshape(NC, NP, NS, NCH1, CH)
    src1 = jnp.concatenate(
        [src1, jnp.zeros((NC, NP, NS, 1, CH), jnp.int32)], axis=3)
    dst1 = jnp.stack([
        jnp.concatenate([dst[:e1], pad_dst1]),
        jnp.concatenate([dst[e1:], pad_dst1]),
    ]).reshape(NC, NS, NCH1, CH)

    tm = NS * NCHM * CH
    pm = tm - E
    pad_srcm = jnp.arange(pm, dtype=jnp.int32) % N
    pad_dstm = N + jnp.arange(pm, dtype=jnp.int32) % (AGG_ROWS - N)
    srcm_base = jnp.concatenate([src, pad_srcm])
    srcm = jnp.stack(
        [jnp.stack([2 * (srcm_base + c * N) + p for p in range(NP)])
         for c in range(NC)]
    ).reshape(NC, NP, NS, NCHM, CH)
    srcm = jnp.concatenate(
        [srcm, jnp.zeros((NC, NP, NS, 1, CH), jnp.int32)], axis=3)
    dstm_1 = jnp.concatenate([dst, pad_dstm])
    dstm = jnp.stack([dstm_1, dstm_1]).reshape(NC, NS, NCHM, CH)

    batch2d = batch.astype(jnp.int32).reshape(N, 1)

    p = params['layers'][0]
    sc = (1.0 + p['eps']).reshape(1, 1).astype(jnp.float32)
    agg = _agg_l1(x.reshape(NP * N, DQ), src1, dst1)
    h = _mlp_l1(x, agg, sc, p['W1'], p['b1'].reshape(1, DHID),
                p['W2'], p['b2'].reshape(1, DHID))

    for p in params['layers'][1:]:
        sc = (1.0 + p['eps']).reshape(1, 1).astype(jnp.float32)
        table = h.reshape(NC * NP * N, DQ)
        agg = _agg_mid(table, srcm, dstm)
        h = _mlp_mid(h, agg, sc, p['W1'], p['b1'].reshape(1, DHID),
                     p['W2'], p['b2'].reshape(1, DHID))

    return _pool(h, batch2d, params['Wout'], params['bout'].reshape(1, 1))
